# baseline (speedup 1.0000x reference)
import functools

import jax
import jax.numpy as jnp
from jax import lax
from jax.experimental import pallas as pl
from jax.experimental.pallas import tpu as pltpu
from jax.experimental.pallas import tpu_sc as plsc

TOKENS = 16384
D_MODEL = 2048
NUM_EXPERTS = 64
TOP_K = 2
BT = 2048
NCHUNK = 4
CTOK = TOKENS // NCHUNK

_SC_INFO = plsc.get_sparse_core_info()
_NC = _SC_INFO.num_cores
_NS = _SC_INFO.num_subcores
_L = _SC_INFO.num_lanes
_NW = _NC * _NS
_CHUNK = CTOK // _NW
_NGROUPS = _CHUNK // _L


def _router_tc_body(x_ref, w_ref, scores_ref, logits_ref):
    x = x_ref[...]
    w = w_ref[...]
    logits = jax.lax.dot_general(
        x, w, (((1,), (1,)), ((), ())), preferred_element_type=jnp.float32
    )
    m = jnp.max(logits, axis=-1, keepdims=True)
    e = jnp.exp(logits - m)
    s = jnp.sum(e, axis=-1, keepdims=True)
    logits_ref[...] = logits
    scores_ref[...] = e / s


def _dense_stage(x, W, c):
    grid = (CTOK // BT,)
    out_shapes = (
        jax.ShapeDtypeStruct((CTOK, NUM_EXPERTS), jnp.float32),
        jax.ShapeDtypeStruct((CTOK, NUM_EXPERTS), jnp.float32),
    )
    base_blk = c * (CTOK // BT)
    return pl.pallas_call(
        _router_tc_body,
        grid=grid,
        in_specs=[
            pl.BlockSpec((BT, D_MODEL), lambda i: (i + base_blk, 0)),
            pl.BlockSpec((NUM_EXPERTS, D_MODEL), lambda i: (0, 0)),
        ],
        out_specs=[
            pl.BlockSpec((BT, NUM_EXPERTS), lambda i: (i, 0)),
            pl.BlockSpec((BT, NUM_EXPERTS), lambda i: (i, 0)),
        ],
        out_shape=out_shapes,
        compiler_params=pltpu.CompilerParams(
            dimension_semantics=("arbitrary",),
        ),
    )(x, W)


def _top2_sc_kernel(scores_hbm, ew_hbm, ei_hbm, buf, ew_buf, ei_buf):
    wid = lax.axis_index("s") * _NC + lax.axis_index("c")
    base = wid * _CHUNK
    pltpu.sync_copy(
        scores_hbm.at[pl.ds(base * NUM_EXPERTS, _CHUNK * NUM_EXPERTS)], buf
    )

    lane = lax.iota(jnp.int32, _L)
    neg_inf = jnp.full((_L,), -jnp.inf, jnp.float32)
    zero_i = jnp.zeros((_L,), jnp.int32)

    def group_body(g, carry):
        tok = g * _L + lane
        tok64 = tok * NUM_EXPERTS
        m1, m2 = neg_inf, neg_inf
        i1, i2 = zero_i, zero_i
        for e in range(NUM_EXPERTS):
            e_i = (lane + e) & (NUM_EXPERTS - 1)
            v = plsc.load_gather(buf, [tok64 + e_i])
            gt1 = v > m1
            gt2 = v > m2
            n_i2 = jnp.where(gt1, i1, jnp.where(gt2, e_i, i2))
            n_m2 = jnp.where(gt1, m1, jnp.where(gt2, v, m2))
            i1 = jnp.where(gt1, e_i, i1)
            m1 = jnp.where(gt1, v, m1)
            i2, m2 = n_i2, n_m2
        two_tok = tok * TOP_K
        plsc.store_scatter(ew_buf, [two_tok], m1)
        plsc.store_scatter(ew_buf, [two_tok + 1], m2)
        plsc.store_scatter(ei_buf, [two_tok], i1)
        plsc.store_scatter(ei_buf, [two_tok + 1], i2)
        return carry

    lax.fori_loop(0, _NGROUPS, group_body, 0)
    pltpu.sync_copy(ew_buf, ew_hbm.at[pl.ds(base * TOP_K, _CHUNK * TOP_K)])
    pltpu.sync_copy(ei_buf, ei_hbm.at[pl.ds(base * TOP_K, _CHUNK * TOP_K)])


@functools.partial(
    pl.kernel,
    mesh=plsc.VectorSubcoreMesh(core_axis_name="c", subcore_axis_name="s"),
    out_type=[
        jax.ShapeDtypeStruct((CTOK * TOP_K,), jnp.float32),
        jax.ShapeDtypeStruct((CTOK * TOP_K,), jnp.int32),
    ],
    scratch_types=[
        pltpu.VMEM((_CHUNK * NUM_EXPERTS,), jnp.float32),
        pltpu.VMEM((_CHUNK * TOP_K,), jnp.float32),
        pltpu.VMEM((_CHUNK * TOP_K,), jnp.int32),
    ],
    compiler_params=pltpu.CompilerParams(needs_layout_passes=False),
)
def _top2_stage(scores_flat, ew_flat, ei_flat, buf, ew_buf, ei_buf):
    _top2_sc_kernel(scores_flat, ew_flat, ei_flat, buf, ew_buf, ei_buf)


@jax.jit
def kernel(x, W):
    scores_c, logits_c, ew_c, ei_c = [], [], [], []
    for c in range(NCHUNK):
        s, l = _dense_stage(x, W, c)
        ewf, eif = _top2_stage(s.reshape(-1))
        scores_c.append(s)
        logits_c.append(l)
        ew_c.append(ewf.reshape(CTOK, TOP_K))
        ei_c.append(eif.reshape(CTOK, TOP_K))
    scores = jnp.concatenate(scores_c, axis=0)
    logits = jnp.concatenate(logits_c, axis=0)
    ew = jnp.concatenate(ew_c, axis=0)
    ei = jnp.concatenate(ei_c, axis=0)
    return scores, logits, ew, ei


# speedup vs baseline: 1.8262x; 1.8262x over previous
import jax
import jax.numpy as jnp
from jax.experimental import pallas as pl
from jax.experimental.pallas import tpu as pltpu

TOKENS = 16384
D_MODEL = 2048
NUM_EXPERTS = 64
TOP_K = 2
BT = 2048
DH = D_MODEL // 2


def _router_body(x1_ref, x2_ref, w_ref, scores_ref, logits_ref, ew_ref, ei_ref):
    w = w_ref[...]
    logits = jax.lax.dot_general(
        x1_ref[...], w[:, :DH], (((1,), (1,)), ((), ())),
        preferred_element_type=jnp.float32,
    )
    logits += jax.lax.dot_general(
        x2_ref[...], w[:, DH:], (((1,), (1,)), ((), ())),
        preferred_element_type=jnp.float32,
    )
    m1 = jnp.max(logits, axis=-1, keepdims=True)
    e = jnp.exp(logits - m1)
    s = jnp.sum(e, axis=-1, keepdims=True)
    logits_ref[...] = logits
    scores_ref[...] = e / s

    iota = jax.lax.broadcasted_iota(jnp.int32, logits.shape, 1)
    i1 = jnp.min(jnp.where(logits == m1, iota, NUM_EXPERTS), axis=-1, keepdims=True)
    masked = jnp.where(iota == i1, -jnp.inf, logits)
    m2 = jnp.max(masked, axis=-1, keepdims=True)
    i2 = jnp.min(jnp.where(masked == m2, iota, NUM_EXPERTS), axis=-1, keepdims=True)
    ew_ref[:, 0:1] = 1.0 / s
    ew_ref[:, 1:2] = jnp.exp(m2 - m1) / s
    ei_ref[:, 0:1] = i1
    ei_ref[:, 1:2] = i2


@jax.jit
def kernel(x, W):
    grid = (TOKENS // BT,)
    out_shapes = (
        jax.ShapeDtypeStruct((TOKENS, NUM_EXPERTS), jnp.float32),
        jax.ShapeDtypeStruct((TOKENS, NUM_EXPERTS), jnp.float32),
        jax.ShapeDtypeStruct((TOKENS, TOP_K), jnp.float32),
        jax.ShapeDtypeStruct((TOKENS, TOP_K), jnp.int32),
    )
    scores, logits, ew, ei = pl.pallas_call(
        _router_body,
        grid=grid,
        in_specs=[
            pl.BlockSpec((BT, DH), lambda i: (i, 0)),
            pl.BlockSpec((BT, DH), lambda i: (i, 1)),
            pl.BlockSpec((NUM_EXPERTS, D_MODEL), lambda i: (0, 0)),
        ],
        out_specs=[
            pl.BlockSpec((BT, NUM_EXPERTS), lambda i: (i, 0)),
            pl.BlockSpec((BT, NUM_EXPERTS), lambda i: (i, 0)),
            pl.BlockSpec((BT, TOP_K), lambda i: (i, 0)),
            pl.BlockSpec((BT, TOP_K), lambda i: (i, 0)),
        ],
        out_shape=out_shapes,
        compiler_params=pltpu.CompilerParams(
            dimension_semantics=("arbitrary",),
        ),
    )(x, x, W)
    return scores, logits, ew, ei
